# Initial kernel scaffold; baseline (speedup 1.0000x reference)
import jax
import jax.numpy as jnp
from jax.experimental import pallas as pl


def kernel(x, edge_index, W1l, b1, W1r, W2l, b2, W2r):
    raise NotImplementedError("write your pallas kernel here")


# speedup vs baseline: 12.3548x; 12.3548x over previous
import functools

import jax
import jax.numpy as jnp
from jax import lax
from jax.experimental import pallas as pl
from jax.experimental.pallas import tpu as pltpu
from jax.experimental.pallas import tpu_sc as plsc

NC = 2
NS = 16
NW = NC * NS
CHUNK = 128


def _make_sc_agg(n_pad, n_chunks, with_cnt):
  rpt = n_pad // NS
  mesh = plsc.VectorSubcoreMesh(core_axis_name="c", subcore_axis_name="s")

  out_type = [jax.ShapeDtypeStruct((NC, n_pad, 16), jnp.float32)]
  scratch = [
      pltpu.VMEM((n_chunks, CHUNK), jnp.int32),
      pltpu.VMEM((n_chunks, CHUNK), jnp.int32),
      pltpu.VMEM((CHUNK, 16), jnp.float32),
      pltpu.VMEM((rpt, 16), jnp.float32),
      pltpu.VMEM_SHARED((n_pad, 16), jnp.float32),
      pltpu.SemaphoreType.DMA,
  ]
  if with_cnt:
    out_type.append(jax.ShapeDtypeStruct((NC, n_pad, 16), jnp.float32))
    scratch += [
        pltpu.VMEM((CHUNK, 16), jnp.float32),
        pltpu.VMEM_SHARED((n_pad, 16), jnp.float32),
    ]

  def body(table, src3, dst3, *rest):
    if with_cnt:
      (out_f, out_c, src_v, dst_v, rows_v, stripe_v, acc_s, sem,
       ones_v, cnt_s) = rest
    else:
      out_f, src_v, dst_v, rows_v, stripe_v, acc_s, sem = rest
      out_c = ones_v = cnt_s = None
    c = lax.axis_index("c")
    s = lax.axis_index("s")
    w = s * NC + c

    def zrow(i, _):
      stripe_v[i, :] = jnp.zeros((16,), jnp.float32)
      return _
    lax.fori_loop(0, rpt, zrow, None)
    pltpu.sync_copy(stripe_v, acc_s.at[pl.ds(s * rpt, rpt)])
    if with_cnt:
      pltpu.sync_copy(stripe_v, cnt_s.at[pl.ds(s * rpt, rpt)])

      def orow(i, _):
        ones_v[i, :] = jnp.ones((16,), jnp.float32)
        return _
      lax.fori_loop(0, CHUNK, orow, None)

    pltpu.sync_copy(src3.at[w], src_v)
    pltpu.sync_copy(dst3.at[w], dst_v)
    plsc.subcore_barrier()

    def chunk(ci, _):
      pltpu.async_copy(table.at[src_v.at[ci]], rows_v, sem).wait()
      pltpu.sync_copy(rows_v, acc_s.at[dst_v.at[ci]], add=True)
      if with_cnt:
        pltpu.sync_copy(ones_v, cnt_s.at[dst_v.at[ci]], add=True)
      return _
    lax.fori_loop(0, n_chunks, chunk, None)
    plsc.subcore_barrier()

    sl = pl.ds(s * rpt, rpt)
    pltpu.sync_copy(acc_s.at[sl], out_f.at[c, sl])
    if with_cnt:
      pltpu.sync_copy(cnt_s.at[sl], out_c.at[c, sl])

  return pl.kernel(
      body, out_type, mesh=mesh, scratch_types=scratch,
      compiler_params=pltpu.CompilerParams(use_tc_tiling_on_sc=False))


def _tc1_body(x_ref, wl_ref, wr_ref, pl_ref, pr_ref):
  x = x_ref[...]
  pl_ref[...] = jnp.dot(x, wl_ref[...], preferred_element_type=jnp.float32)
  pr_ref[...] = jnp.dot(x, wr_ref[...], preferred_element_type=jnp.float32)


def _tc2_body(accf_ref, accc_ref, xr_ref, b1_ref, wl_ref, wr_ref,
              hp_ref, hr_ref, cnt_ref):
  f = accf_ref[0] + accf_ref[1]
  cnt = accc_ref[0] + accc_ref[1]
  mean = f / jnp.maximum(cnt, 1.0)
  h = jnp.maximum(mean + b1_ref[...] + xr_ref[...], 0.0)
  hp_ref[...] = jnp.dot(h, wl_ref[...], preferred_element_type=jnp.float32)
  hr_ref[...] = jnp.dot(h, wr_ref[...], preferred_element_type=jnp.float32)
  cnt_ref[...] = cnt


def _tc3_body(accf_ref, cnt_ref, hr_ref, b2_ref, out_ref):
  f = accf_ref[0] + accf_ref[1]
  z = f / jnp.maximum(cnt_ref[...], 1.0) + b2_ref[...] + hr_ref[...]
  m = jnp.max(z, axis=1, keepdims=True)
  zs = z - m
  out_ref[...] = zs - jnp.log(jnp.sum(jnp.exp(zs), axis=1, keepdims=True))


def kernel(x, edge_index, W1l, b1, W1r, W2l, b2, W2r):
  n, f_in = x.shape
  e = edge_index.shape[1]
  h = W1l.shape[1]
  assert h == 16 and W2l.shape[1] == 16

  n_pad = 10240
  assert n <= n_pad - 1
  blk = 1024
  grid = n_pad // blk

  per_tile = -(-e // NW)
  n_chunks = -(-per_tile // CHUNK)
  e_pad = NW * n_chunks * CHUNK
  src = edge_index[0]
  dst = edge_index[1]
  pad = e_pad - e
  src3 = jnp.concatenate(
      [src, jnp.zeros((pad,), jnp.int32)]).reshape(NW, n_chunks, CHUNK)
  dst3 = jnp.concatenate(
      [dst, jnp.full((pad,), n, jnp.int32)]).reshape(NW, n_chunks, CHUNK)

  x_pad = jnp.pad(x, ((0, n_pad - n), (0, 0)))
  b1r = b1.reshape(1, h)
  b2r = b2.reshape(1, W2l.shape[1])

  xp1, xr1 = pl.pallas_call(
      _tc1_body,
      grid=(grid,),
      in_specs=[
          pl.BlockSpec((blk, f_in), lambda i: (i, 0)),
          pl.BlockSpec((f_in, h), lambda i: (0, 0)),
          pl.BlockSpec((f_in, h), lambda i: (0, 0)),
      ],
      out_specs=[pl.BlockSpec((blk, h), lambda i: (i, 0))] * 2,
      out_shape=[jax.ShapeDtypeStruct((n_pad, h), jnp.float32)] * 2,
  )(x_pad, W1l, W1r)

  agg1 = _make_sc_agg(n_pad, n_chunks, with_cnt=True)
  acc1, acc_cnt = agg1(xp1, src3, dst3)

  full_spec = pl.BlockSpec((blk, h), lambda i: (i, 0))
  acc_spec = pl.BlockSpec((NC, blk, h), lambda i: (0, i, 0))
  w_spec = pl.BlockSpec((h, h), lambda i: (0, 0))
  b_spec = pl.BlockSpec((1, h), lambda i: (0, 0))
  hp2, hr2, cnt16 = pl.pallas_call(
      _tc2_body,
      grid=(grid,),
      in_specs=[acc_spec, acc_spec, full_spec, b_spec, w_spec, w_spec],
      out_specs=[full_spec] * 3,
      out_shape=[jax.ShapeDtypeStruct((n_pad, h), jnp.float32)] * 3,
  )(acc1, acc_cnt, xr1, b1r, W2l, W2r)

  agg2 = _make_sc_agg(n_pad, n_chunks, with_cnt=False)
  (acc2,) = agg2(hp2, src3, dst3)

  out = pl.pallas_call(
      _tc3_body,
      grid=(grid,),
      in_specs=[acc_spec, full_spec, full_spec, b_spec],
      out_specs=full_spec,
      out_shape=jax.ShapeDtypeStruct((n_pad, h), jnp.float32),
  )(acc2, cnt16, hr2, b2r)

  return out[:n]


# speedup vs baseline: 14.9880x; 1.2131x over previous
import functools

import jax
import jax.numpy as jnp
from jax import lax
from jax.experimental import pallas as pl
from jax.experimental.pallas import tpu as pltpu
from jax.experimental.pallas import tpu_sc as plsc

NC = 2
NS = 16
NW = NC * NS
CHUNK = 128
KB = 8


def _make_sc_agg(n_pad, n_blocks, with_cnt):
  rpt = n_pad // NS
  mesh = plsc.VectorSubcoreMesh(core_axis_name="c", subcore_axis_name="s")

  out_type = [jax.ShapeDtypeStruct((NC, n_pad, 16), jnp.float32)]
  scratch = [
      pltpu.VMEM((n_blocks, KB * CHUNK), jnp.int32),
      pltpu.VMEM((n_blocks, KB * CHUNK), jnp.int32),
      pltpu.VMEM((KB * CHUNK, 16), jnp.float32),
      pltpu.VMEM((KB * CHUNK, 16), jnp.float32),
      pltpu.VMEM((rpt, 16), jnp.float32),
      pltpu.VMEM_SHARED((n_pad, 16), jnp.float32),
      pltpu.SemaphoreType.DMA,
      pltpu.SemaphoreType.DMA,
  ]
  if with_cnt:
    out_type.append(jax.ShapeDtypeStruct((NC, n_pad, 16), jnp.float32))
    scratch += [
        pltpu.VMEM((KB * CHUNK, 16), jnp.float32),
        pltpu.VMEM_SHARED((n_pad, 16), jnp.float32),
    ]

  def body(table, src4, dst4, *rest):
    if with_cnt:
      (out_f, out_c, src_v, dst_v, rows0, rows1, stripe_v, acc_s,
       sem0, sem1, ones_v, cnt_s) = rest
    else:
      (out_f, src_v, dst_v, rows0, rows1, stripe_v, acc_s,
       sem0, sem1) = rest
      out_c = ones_v = cnt_s = None
    c = lax.axis_index("c")
    s = lax.axis_index("s")
    w = s * NC + c

    def zrow(i, _):
      stripe_v[i, :] = jnp.zeros((16,), jnp.float32)
      return _
    lax.fori_loop(0, rpt, zrow, None)
    pltpu.sync_copy(stripe_v, acc_s.at[pl.ds(s * rpt, rpt)])
    if with_cnt:
      pltpu.sync_copy(stripe_v, cnt_s.at[pl.ds(s * rpt, rpt)])

      def orow(i, _):
        ones_v[i, :] = jnp.ones((16,), jnp.float32)
        return _
      lax.fori_loop(0, KB * CHUNK, orow, None)

    pltpu.sync_copy(src4.at[w], src_v)
    pltpu.sync_copy(dst4.at[w], dst_v)
    plsc.subcore_barrier()

    def gather(bi, rows, sem):
      return pltpu.make_async_copy(table.at[src_v.at[bi]], rows, sem)

    bufs = ((rows0, sem0), (rows1, sem1))
    gather(0, rows0, sem0).start()

    def block_pair(pi, _):
      for b, (rows, sem) in enumerate(bufs):
        bi = pi * 2 + b
        nrows, nsem = bufs[1 - b]

        @pl.when(bi + 1 < n_blocks)
        def _():
          gather(bi + 1, nrows, nsem).start()

        gather(bi, rows, sem).wait()
        pltpu.sync_copy(rows, acc_s.at[dst_v.at[bi]], add=True)
        if with_cnt:
          pltpu.sync_copy(ones_v, cnt_s.at[dst_v.at[bi]], add=True)
      return _
    lax.fori_loop(0, n_blocks // 2, block_pair, None)
    plsc.subcore_barrier()

    sl = pl.ds(s * rpt, rpt)
    pltpu.sync_copy(acc_s.at[sl], out_f.at[c, sl])
    if with_cnt:
      pltpu.sync_copy(cnt_s.at[sl], out_c.at[c, sl])

  return pl.kernel(
      body, out_type, mesh=mesh, scratch_types=scratch,
      compiler_params=pltpu.CompilerParams(use_tc_tiling_on_sc=False))


def _tc1_body(x_ref, wl_ref, wr_ref, pl_ref, pr_ref):
  x = x_ref[...]
  pl_ref[...] = jnp.dot(x, wl_ref[...], preferred_element_type=jnp.float32)
  pr_ref[...] = jnp.dot(x, wr_ref[...], preferred_element_type=jnp.float32)


def _tc2_body(accf_ref, accc_ref, xr_ref, b1_ref, wl_ref, wr_ref,
              hp_ref, hr_ref, cnt_ref):
  f = accf_ref[0] + accf_ref[1]
  cnt = accc_ref[0] + accc_ref[1]
  mean = f / jnp.maximum(cnt, 1.0)
  h = jnp.maximum(mean + b1_ref[...] + xr_ref[...], 0.0)
  hp_ref[...] = jnp.dot(h, wl_ref[...], preferred_element_type=jnp.float32)
  hr_ref[...] = jnp.dot(h, wr_ref[...], preferred_element_type=jnp.float32)
  cnt_ref[...] = cnt


def _tc3_body(accf_ref, cnt_ref, hr_ref, b2_ref, out_ref):
  f = accf_ref[0] + accf_ref[1]
  z = f / jnp.maximum(cnt_ref[...], 1.0) + b2_ref[...] + hr_ref[...]
  m = jnp.max(z, axis=1, keepdims=True)
  zs = z - m
  out_ref[...] = zs - jnp.log(jnp.sum(jnp.exp(zs), axis=1, keepdims=True))


def kernel(x, edge_index, W1l, b1, W1r, W2l, b2, W2r):
  n, f_in = x.shape
  e = edge_index.shape[1]
  h = W1l.shape[1]
  assert h == 16 and W2l.shape[1] == 16

  n_pad = 10240
  assert n <= n_pad - 1
  blk = 1024
  grid = n_pad // blk

  per_tile = -(-e // NW)
  bsz = KB * CHUNK
  n_blocks = -(-per_tile // bsz)
  if n_blocks % 2:
    n_blocks += 1
  e_pad = NW * n_blocks * bsz
  src = edge_index[0]
  dst = edge_index[1]
  pad = e_pad - e
  src3 = jnp.concatenate(
      [src, jnp.zeros((pad,), jnp.int32)]).reshape(NW, n_blocks, KB * CHUNK)
  dst3 = jnp.concatenate(
      [dst, jnp.full((pad,), n, jnp.int32)]).reshape(NW, n_blocks, KB * CHUNK)

  x_pad = jnp.pad(x, ((0, n_pad - n), (0, 0)))
  b1r = b1.reshape(1, h)
  b2r = b2.reshape(1, W2l.shape[1])

  xp1, xr1 = pl.pallas_call(
      _tc1_body,
      grid=(grid,),
      in_specs=[
          pl.BlockSpec((blk, f_in), lambda i: (i, 0)),
          pl.BlockSpec((f_in, h), lambda i: (0, 0)),
          pl.BlockSpec((f_in, h), lambda i: (0, 0)),
      ],
      out_specs=[pl.BlockSpec((blk, h), lambda i: (i, 0))] * 2,
      out_shape=[jax.ShapeDtypeStruct((n_pad, h), jnp.float32)] * 2,
  )(x_pad, W1l, W1r)

  agg1 = _make_sc_agg(n_pad, n_blocks, with_cnt=True)
  acc1, acc_cnt = agg1(xp1, src3, dst3)

  full_spec = pl.BlockSpec((blk, h), lambda i: (i, 0))
  acc_spec = pl.BlockSpec((NC, blk, h), lambda i: (0, i, 0))
  w_spec = pl.BlockSpec((h, h), lambda i: (0, 0))
  b_spec = pl.BlockSpec((1, h), lambda i: (0, 0))
  hp2, hr2, cnt16 = pl.pallas_call(
      _tc2_body,
      grid=(grid,),
      in_specs=[acc_spec, acc_spec, full_spec, b_spec, w_spec, w_spec],
      out_specs=[full_spec] * 3,
      out_shape=[jax.ShapeDtypeStruct((n_pad, h), jnp.float32)] * 3,
  )(acc1, acc_cnt, xr1, b1r, W2l, W2r)

  agg2 = _make_sc_agg(n_pad, n_blocks, with_cnt=False)
  (acc2,) = agg2(hp2, src3, dst3)

  out = pl.pallas_call(
      _tc3_body,
      grid=(grid,),
      in_specs=[acc_spec, full_spec, full_spec, b_spec],
      out_specs=full_spec,
      out_shape=jax.ShapeDtypeStruct((n_pad, h), jnp.float32),
  )(acc2, cnt16, hr2, b2r)

  return out[:n]
